# baseline (speedup 1.0000x reference)
import functools

import jax
import jax.numpy as jnp
from jax import lax
from jax.experimental import pallas as pl
from jax.experimental.pallas import tpu as pltpu
from jax.experimental.pallas import tpu_sc as plsc

N_NODES = 10000
D = 128
N_EDGES = 320000
N_GRAPHS = 64

NC = 2
NS = 16
LANES = 16
EB = 64
NW = NC * NS
TPW = 79
E_PAD = NW * TPW * 2 * EB
ROWS_PER_SUB = 632
N_PAD = ROWS_PER_SUB * NS


def _spmm_body(h_hbm, pk_hbm, ew_hbm, out_hbm,
               pk_all, ew_all, src_u0, dst_u0, src_u1, dst_u1,
               rows0, rows1, agg_sh, sem_g0, sem_g1, sem_s0, sem_s1):
    c = lax.axis_index("c")
    s = lax.axis_index("s")

    def _zrow(r, carry):
        for j in range(D // LANES):
            rows0[r, pl.ds(j * LANES, LANES)] = jnp.zeros((LANES,), jnp.float32)
        return carry
    lax.fori_loop(0, EB, _zrow, 0)
    row0 = s * ROWS_PER_SUB
    nz = ROWS_PER_SUB // EB
    for k in range(nz):
        pltpu.sync_copy(rows0, agg_sh.at[pl.ds(row0 + k * EB, EB)])
    rem = ROWS_PER_SUB - nz * EB
    if rem:
        pltpu.sync_copy(rows0.at[pl.ds(0, rem)],
                        agg_sh.at[pl.ds(row0 + nz * EB, rem)])
    plsc.subcore_barrier()

    wid = c * NS + s
    pltpu.sync_copy(pk_hbm.at[wid], pk_all)
    pltpu.sync_copy(ew_hbm.at[wid], ew_all)

    rows = (rows0, rows1)
    src_u = (src_u0, src_u1)
    dst_u = (dst_u0, dst_u1)
    sem_g = (sem_g0, sem_g1)
    sem_s = (sem_s0, sem_s1)

    def unpack(r, col, u):
        for g in range(EB // LANES):
            sl = pl.ds(g * LANES, LANES)
            p16 = pk_all[r, pl.ds(col + g * LANES, LANES)]
            src_u[u][sl] = lax.shift_right_logical(p16, 14)
            dst_u[u][sl] = jnp.bitwise_and(p16, 16383)

    def fire_gth(u):
        pltpu.async_copy(h_hbm.at[src_u[u]], rows[u], sem_g[u])

    def wait_gth(u):
        pltpu.make_async_copy(h_hbm.at[src_u[u]], rows[u], sem_g[u]).wait()

    def fire_sct(u):
        pltpu.async_copy(rows[u], agg_sh.at[dst_u[u]], sem_s[u], add=True)

    def wait_sct(u):
        pltpu.make_async_copy(rows[u], agg_sh.at[dst_u[u]], sem_s[u]).wait()

    def scale(r, col, u):
        rv = rows[u]

        def _egroup(g, carry2):
            ew16 = ew_all[r, pl.ds(col + g * LANES, LANES)]
            for l in range(LANES):
                idx = jnp.full((LANES,), l, jnp.int32)
                w = lax.gather(
                    ew16, idx[:, None],
                    lax.GatherDimensionNumbers(offset_dims=(),
                                               collapsed_slice_dims=(0,),
                                               start_index_map=(0,)),
                    (1,), mode=lax.GatherScatterMode.PROMISE_IN_BOUNDS)
                e = g * LANES + l
                for j in range(D // LANES):
                    sl = pl.ds(j * LANES, LANES)
                    rv[e, sl] = rv[e, sl] * w
            return carry2
        lax.fori_loop(0, EB // LANES, _egroup, 0)

    unpack(0, 0, 0)
    fire_gth(0)

    def _pair(t, carry):
        @pl.when(t > 0)
        def _():
            wait_sct(1)
        unpack(t, EB, 1)
        fire_gth(1)
        wait_gth(0)
        scale(t, 0, 0)
        fire_sct(0)
        wait_gth(1)
        scale(t, EB, 1)
        wait_sct(0)
        unpack(jnp.minimum(t + 1, TPW - 1), 0, 0)
        fire_gth(0)
        fire_sct(1)
        return carry

    lax.fori_loop(0, TPW, _pair, 0)
    wait_sct(1)
    wait_gth(0)
    plsc.subcore_barrier()

    pltpu.sync_copy(agg_sh.at[pl.ds(row0, ROWS_PER_SUB)],
                    out_hbm.at[c, pl.ds(row0, ROWS_PER_SUB)])


_spmm = functools.partial(
    pl.kernel,
    out_type=jax.ShapeDtypeStruct((NC, N_PAD, D), jnp.float32),
    mesh=plsc.VectorSubcoreMesh(core_axis_name="c", subcore_axis_name="s"),
    scratch_types=[
        pltpu.VMEM((TPW, 2 * EB), jnp.int32),
        pltpu.VMEM((TPW, 2 * EB), jnp.float32),
        pltpu.VMEM((EB,), jnp.int32),
        pltpu.VMEM((EB,), jnp.int32),
        pltpu.VMEM((EB,), jnp.int32),
        pltpu.VMEM((EB,), jnp.int32),
        pltpu.VMEM((EB, D), jnp.float32),
        pltpu.VMEM((EB, D), jnp.float32),
        pltpu.VMEM_SHARED((N_PAD, D), jnp.float32),
        pltpu.SemaphoreType.DMA,
        pltpu.SemaphoreType.DMA,
        pltpu.SemaphoreType.DMA,
        pltpu.SemaphoreType.DMA,
    ],
)(_spmm_body)


def _layer_body(agg_ref, h_ref, wr_ref, ws_ref, b_ref, g_ref, be_ref, o_ref):
    agg = agg_ref[0, :N_NODES, :] + agg_ref[1, :N_NODES, :]
    pre = jnp.dot(agg, wr_ref[...], preferred_element_type=jnp.float32)
    pre = pre + jnp.dot(h_ref[...], ws_ref[...], preferred_element_type=jnp.float32)
    pre = pre + b_ref[...]
    mu = jnp.mean(pre, axis=0, keepdims=True)
    var = jnp.mean((pre - mu) ** 2, axis=0, keepdims=True)
    xn = (pre - mu) * lax.rsqrt(var + 1e-5)
    o_ref[...] = jnp.maximum(xn * g_ref[...] + be_ref[...], 0.0)


_layer = pl.pallas_call(
    _layer_body,
    out_shape=jax.ShapeDtypeStruct((N_NODES, D), jnp.float32),
)


def _head_body(h_ref, batch_ref, wnf_ref, bnf_ref, wfc_ref, bfc_ref, o_ref):
    gids = lax.broadcasted_iota(jnp.int32, (N_GRAPHS, 1), 0)
    onehot = (batch_ref[...] == gids).astype(jnp.float32)
    pooled = jnp.dot(onehot, h_ref[...], preferred_element_type=jnp.float32)
    pooled = jnp.dot(pooled, wnf_ref[...], preferred_element_type=jnp.float32) + bnf_ref[...]
    o_ref[...] = jnp.dot(pooled, wfc_ref[...], preferred_element_type=jnp.float32) + bfc_ref[...]


_head = pl.pallas_call(
    _head_body,
    out_shape=jax.ShapeDtypeStruct((N_GRAPHS, 16), jnp.float32),
)


def kernel(x, edge_index, edge_attr, batch,
           Wr1, Ws1, b1, g1, be1,
           Wr2, Ws2, b2, g2, be2,
           Wr3, Ws3, b3, g3, be3,
           W_nf, b_nf, W_fc, b_fc):
    ei = edge_index.astype(jnp.int32)
    pad = E_PAD - N_EDGES
    pidx = jnp.arange(pad, dtype=jnp.int32)
    srcp = jnp.concatenate([ei[0], pidx % N_NODES])
    dstp = jnp.concatenate([ei[1], N_NODES + pidx % (N_PAD - N_NODES)])
    pk = ((srcp << 14) | dstp).reshape(NW, TPW, 2 * EB)
    ew2 = jnp.pad(edge_attr, (0, pad)).reshape(NW, TPW, 2 * EB)

    batch2d = batch.astype(jnp.int32).reshape(1, N_NODES)
    h = x
    for Wr, Ws, b, g, be in ((Wr1, Ws1, b1, g1, be1),
                             (Wr2, Ws2, b2, g2, be2),
                             (Wr3, Ws3, b3, g3, be3)):
        agg = _spmm(h, pk, ew2)
        h = _layer(agg, h, Wr, Ws, b.reshape(1, D), g.reshape(1, D),
                   be.reshape(1, D))
    return _head(h, batch2d, W_nf, b_nf.reshape(1, D), W_fc,
                 b_fc.reshape(1, 16))


# speedup vs baseline: 1.0006x; 1.0006x over previous
import functools

import jax
import jax.numpy as jnp
from jax import lax
from jax.experimental import pallas as pl
from jax.experimental.pallas import tpu as pltpu
from jax.experimental.pallas import tpu_sc as plsc

N_NODES = 10000
D = 128
N_EDGES = 320000
N_GRAPHS = 64

NC = 2
NS = 16
LANES = 16
EB = 64
NW = NC * NS
TPW = 79
E_PAD = NW * TPW * 2 * EB
ROWS_PER_SUB = 632
N_PAD = ROWS_PER_SUB * NS


def _spmm_body(h_hbm, pk_hbm, ew_hbm, out_hbm,
               pk_all, ew_all, src_u0, dst_u0, src_u1, dst_u1,
               rows0, rows1, agg_sh, sem_g0, sem_g1, sem_s0, sem_s1):
    c = lax.axis_index("c")
    s = lax.axis_index("s")

    def _zrow(r, carry):
        for j in range(D // LANES):
            rows0[r, pl.ds(j * LANES, LANES)] = jnp.zeros((LANES,), jnp.float32)
        return carry
    lax.fori_loop(0, EB, _zrow, 0)
    row0 = s * ROWS_PER_SUB
    nz = ROWS_PER_SUB // EB
    for k in range(nz):
        pltpu.sync_copy(rows0, agg_sh.at[pl.ds(row0 + k * EB, EB)])
    rem = ROWS_PER_SUB - nz * EB
    if rem:
        pltpu.sync_copy(rows0.at[pl.ds(0, rem)],
                        agg_sh.at[pl.ds(row0 + nz * EB, rem)])
    plsc.subcore_barrier()

    wid = c * NS + s
    pltpu.sync_copy(pk_hbm.at[wid], pk_all)
    pltpu.sync_copy(ew_hbm.at[wid], ew_all)

    rows = (rows0, rows1)
    src_u = (src_u0, src_u1)
    dst_u = (dst_u0, dst_u1)
    sem_g = (sem_g0, sem_g1)
    sem_s = (sem_s0, sem_s1)

    def unpack(r, col, u):
        for g in range(EB // LANES):
            sl = pl.ds(g * LANES, LANES)
            p16 = pk_all[r, pl.ds(col + g * LANES, LANES)]
            src_u[u][sl] = lax.shift_right_logical(p16, 14)
            dst_u[u][sl] = jnp.bitwise_and(p16, 16383)

    def fire_gth(u):
        pltpu.async_copy(h_hbm.at[src_u[u]], rows[u], sem_g[u])

    def wait_gth(u):
        pltpu.make_async_copy(h_hbm.at[src_u[u]], rows[u], sem_g[u]).wait()

    def fire_sct(u):
        pltpu.async_copy(rows[u], agg_sh.at[dst_u[u]], sem_s[u], add=True)

    def wait_sct(u):
        pltpu.make_async_copy(rows[u], agg_sh.at[dst_u[u]], sem_s[u]).wait()

    def scale(r, col, u):
        rv = rows[u]

        def _egroup(g, carry2):
            ew16 = ew_all[r, pl.ds(col + g * LANES, LANES)]
            for l in range(LANES):
                idx = jnp.full((LANES,), l, jnp.int32)
                w = lax.gather(
                    ew16, idx[:, None],
                    lax.GatherDimensionNumbers(offset_dims=(),
                                               collapsed_slice_dims=(0,),
                                               start_index_map=(0,)),
                    (1,), mode=lax.GatherScatterMode.PROMISE_IN_BOUNDS)
                e = g * LANES + l
                for j in range(D // LANES):
                    sl = pl.ds(j * LANES, LANES)
                    rv[e, sl] = rv[e, sl] * w
            return carry2
        lax.fori_loop(0, EB // LANES, _egroup, 0)

    unpack(0, 0, 0)
    fire_gth(0)

    def _pair(t, carry):
        @pl.when(t > 0)
        def _():
            wait_sct(1)
        unpack(t, EB, 1)
        fire_gth(1)
        wait_gth(0)
        scale(t, 0, 0)
        fire_sct(0)
        wait_gth(1)
        scale(t, EB, 1)
        wait_sct(0)
        unpack(jnp.minimum(t + 1, TPW - 1), 0, 0)
        fire_gth(0)
        fire_sct(1)
        return carry

    lax.fori_loop(0, TPW, _pair, 0)
    wait_sct(1)
    wait_gth(0)
    plsc.subcore_barrier()

    pltpu.sync_copy(agg_sh.at[pl.ds(row0, ROWS_PER_SUB)],
                    out_hbm.at[c, pl.ds(row0, ROWS_PER_SUB)])


_spmm = functools.partial(
    pl.kernel,
    out_type=jax.ShapeDtypeStruct((NC, N_PAD, D), jnp.float32),
    mesh=plsc.VectorSubcoreMesh(core_axis_name="c", subcore_axis_name="s"),
    compiler_params=pltpu.CompilerParams(needs_layout_passes=False),
    scratch_types=[
        pltpu.VMEM((TPW, 2 * EB), jnp.int32),
        pltpu.VMEM((TPW, 2 * EB), jnp.float32),
        pltpu.VMEM((EB,), jnp.int32),
        pltpu.VMEM((EB,), jnp.int32),
        pltpu.VMEM((EB,), jnp.int32),
        pltpu.VMEM((EB,), jnp.int32),
        pltpu.VMEM((EB, D), jnp.float32),
        pltpu.VMEM((EB, D), jnp.float32),
        pltpu.VMEM_SHARED((N_PAD, D), jnp.float32),
        pltpu.SemaphoreType.DMA,
        pltpu.SemaphoreType.DMA,
        pltpu.SemaphoreType.DMA,
        pltpu.SemaphoreType.DMA,
    ],
)(_spmm_body)


def _layer_body(agg_ref, h_ref, wr_ref, ws_ref, b_ref, g_ref, be_ref, o_ref):
    agg = agg_ref[0, :N_NODES, :] + agg_ref[1, :N_NODES, :]
    pre = jnp.dot(agg, wr_ref[...], preferred_element_type=jnp.float32)
    pre = pre + jnp.dot(h_ref[...], ws_ref[...], preferred_element_type=jnp.float32)
    pre = pre + b_ref[...]
    mu = jnp.mean(pre, axis=0, keepdims=True)
    var = jnp.mean((pre - mu) ** 2, axis=0, keepdims=True)
    xn = (pre - mu) * lax.rsqrt(var + 1e-5)
    o_ref[...] = jnp.maximum(xn * g_ref[...] + be_ref[...], 0.0)


_layer = pl.pallas_call(
    _layer_body,
    out_shape=jax.ShapeDtypeStruct((N_NODES, D), jnp.float32),
)


def _head_body(h_ref, batch_ref, wnf_ref, bnf_ref, wfc_ref, bfc_ref, o_ref):
    gids = lax.broadcasted_iota(jnp.int32, (N_GRAPHS, 1), 0)
    onehot = (batch_ref[...] == gids).astype(jnp.float32)
    pooled = jnp.dot(onehot, h_ref[...], preferred_element_type=jnp.float32)
    pooled = jnp.dot(pooled, wnf_ref[...], preferred_element_type=jnp.float32) + bnf_ref[...]
    o_ref[...] = jnp.dot(pooled, wfc_ref[...], preferred_element_type=jnp.float32) + bfc_ref[...]


_head = pl.pallas_call(
    _head_body,
    out_shape=jax.ShapeDtypeStruct((N_GRAPHS, 16), jnp.float32),
)


def kernel(x, edge_index, edge_attr, batch,
           Wr1, Ws1, b1, g1, be1,
           Wr2, Ws2, b2, g2, be2,
           Wr3, Ws3, b3, g3, be3,
           W_nf, b_nf, W_fc, b_fc):
    ei = edge_index.astype(jnp.int32)
    pad = E_PAD - N_EDGES
    pidx = jnp.arange(pad, dtype=jnp.int32)
    srcp = jnp.concatenate([ei[0], pidx % N_NODES])
    dstp = jnp.concatenate([ei[1], N_NODES + pidx % (N_PAD - N_NODES)])
    pk = ((srcp << 14) | dstp).reshape(NW, TPW, 2 * EB)
    ew2 = jnp.pad(edge_attr, (0, pad)).reshape(NW, TPW, 2 * EB)

    batch2d = batch.astype(jnp.int32).reshape(1, N_NODES)
    h = x
    for Wr, Ws, b, g, be in ((Wr1, Ws1, b1, g1, be1),
                             (Wr2, Ws2, b2, g2, be2),
                             (Wr3, Ws3, b3, g3, be3)):
        agg = _spmm(h, pk, ew2)
        h = _layer(agg, h, Wr, Ws, b.reshape(1, D), g.reshape(1, D),
                   be.reshape(1, D))
    return _head(h, batch2d, W_nf, b_nf.reshape(1, D), W_fc,
                 b_fc.reshape(1, 16))
